# baseline (speedup 1.0000x reference)
import functools

import jax
import jax.numpy as jnp
from jax import lax
from jax.experimental import pallas as pl
from jax.experimental.pallas import tpu as pltpu
from jax.experimental.pallas import tpu_sc as plsc

N = 10000
E = 320000
D = 128
N_PAD = 10240

NC, NS = 2, 16
NW = NC * NS
EH = E // 2
EPW = EH // NW

EQ = E // 4
EPT = EQ // NS
GCH = 104
NCHUNK_G = (EPT + GCH - 1) // GCH
NPAIR_G = (NCHUNK_G - 1) // 2
QBLKS = EQ // 2000

SCH = 40
NCHUNK_S = EPW // SCH
NPAIR_S = (NCHUNK_S - 1) // 2

NODE_BLK = 2000
EDGE_BLK = 2000
HBLKS = EH // EDGE_BLK

_mesh = plsc.VectorSubcoreMesh(core_axis_name="c", subcore_axis_name="s")


def _proj_body(nf_ref, wa_ref, wb_ref, z_ref):
    nf = nf_ref[...]
    z_ref[0] = jnp.dot(nf, wa_ref[...], preferred_element_type=jnp.float32)
    z_ref[1] = jnp.dot(nf, wb_ref[...], preferred_element_type=jnp.float32)


def _edge_math(gs, gr, ef, w1, b1, w2, b2, sc, bi):
    pre = (gs + gr + b1
           + jnp.dot(ef, w1, preferred_element_type=jnp.float32))
    h = jnp.maximum(pre, 0.0)
    y = jnp.dot(h, w2, preferred_element_type=jnp.float32) + b2
    mean = jnp.mean(y, axis=-1, keepdims=True)
    var = jnp.mean((y - mean) ** 2, axis=-1, keepdims=True)
    upd = (y - mean) * lax.rsqrt(var + 1e-5) * sc + bi
    return upd, upd + ef


def _make_edge_body(n_dummy):
    def body(*refs):
        (gs_ref, gr_ref, ef_ref, w1_ref, b1_ref, w2_ref, b2_ref,
         sc_ref, bi_ref, upd_ref, new_ref) = refs[n_dummy:]
        upd, new = _edge_math(gs_ref[0], gr_ref[0], ef_ref[...], w1_ref[...],
                              b1_ref[...], w2_ref[...], b2_ref[...],
                              sc_ref[...], bi_ref[...])
        upd_ref[...] = upd
        new_ref[...] = new
    return body


def _node_body(nf_ref, a0_ref, a1_ref, a2_ref, a3_ref, w1a_ref, w1b_ref,
               b1_ref, w2_ref, b2_ref, sc_ref, bi_ref, out_ref):
    nf = nf_ref[...]
    agg = (a0_ref[0] + a1_ref[0]) + (a2_ref[0] + a3_ref[0])
    pre = (jnp.dot(nf, w1a_ref[...], preferred_element_type=jnp.float32)
           + jnp.dot(agg, w1b_ref[...], preferred_element_type=jnp.float32)
           + b1_ref[...])
    h = jnp.maximum(pre, 0.0)
    y = jnp.dot(h, w2_ref[...], preferred_element_type=jnp.float32) + b2_ref[...]
    mean = jnp.mean(y, axis=-1, keepdims=True)
    var = jnp.mean((y - mean) ** 2, axis=-1, keepdims=True)
    out_ref[...] = ((y - mean) * lax.rsqrt(var + 1e-5) * sc_ref[...]
                    + bi_ref[...] + nf)


def _row_spec(blk):
    return pl.BlockSpec((blk, D), lambda i: (i, 0))


def _full_spec(shape):
    return pl.BlockSpec(shape, lambda i: tuple(0 for _ in shape))


def _proj(nf, wa, wb):
    blk = 2048
    return pl.pallas_call(
        _proj_body,
        grid=(N_PAD // blk,),
        in_specs=[_row_spec(blk), _full_spec((D, D)), _full_spec((D, D))],
        out_specs=pl.BlockSpec((2, blk, D), lambda i: (0, i, 0)),
        out_shape=jax.ShapeDtypeStruct((2, N_PAD, D), jnp.float32),
    )(nf, wa, wb)


def _off_spec(blk0):
    return pl.BlockSpec((EDGE_BLK, D), lambda i: (blk0 + i, 0))


def _g_spec(plane):
    return pl.BlockSpec((1, EDGE_BLK, D), lambda i: (plane, i, 0))


_TINY = pl.BlockSpec((8, D), lambda i: (0, 0))


def _edge_mlp_q(q, new_prev, upd_prev, gcat, ef, w1, b1, w2, b2, sc, bi):
    first_new = q == 0
    fresh_upd = q % 2 == 0
    wspecs = [_full_spec((D, D)), _full_spec((1, D)), _full_spec((D, D)),
              _full_spec((1, D)), _full_spec((1, D)), _full_spec((1, D))]
    alias_in = []
    aliases = {}
    args = []
    if not first_new:
        aliases[len(alias_in)] = 1
        alias_in.append(_TINY)
        args.append(new_prev)
    if not fresh_upd:
        aliases[len(alias_in)] = 0
        alias_in.append(_TINY)
        args.append(upd_prev)
    upd_blk0 = (q % 2) * QBLKS
    return pl.pallas_call(
        _make_edge_body(len(alias_in)),
        grid=(QBLKS,),
        in_specs=alias_in + [_g_spec(0), _g_spec(1), _off_spec(q * QBLKS)]
        + wspecs,
        out_specs=[_off_spec(upd_blk0), _off_spec(q * QBLKS)],
        out_shape=[jax.ShapeDtypeStruct((EH, D), jnp.float32),
                   jax.ShapeDtypeStruct((E, D), jnp.float32)],
        input_output_aliases=aliases,
    )(*args, gcat, gcat, ef, w1, b1, w2, b2, sc, bi)


def _node_mlp(nf, agg_a, agg_b, w1a, w1b, b1, w2, b2, sc, bi):
    grid = N // NODE_BLK
    aspec = [pl.BlockSpec((1, NODE_BLK, D), lambda i: (0, i, 0)),
             pl.BlockSpec((1, NODE_BLK, D), lambda i: (1, i, 0))]
    return pl.pallas_call(
        _node_body,
        grid=(grid,),
        in_specs=[_row_spec(NODE_BLK)] + aspec + aspec
        + [_full_spec((D, D)), _full_spec((D, D)), _full_spec((1, D)),
           _full_spec((D, D)), _full_spec((1, D)), _full_spec((1, D)),
           _full_spec((1, D))],
        out_specs=_row_spec(NODE_BLK),
        out_shape=jax.ShapeDtypeStruct((N, D), jnp.float32),
    )(nf, agg_a, agg_a, agg_b, agg_b, w1a, w1b, b1, w2, b2, sc, bi)


def _make_gather(e0):
    @functools.partial(
        pl.kernel,
        mesh=_mesh,
        out_type=jax.ShapeDtypeStruct((2, EQ, D), jnp.float32),
        scratch_types=[
            pltpu.VMEM((GCH,), jnp.int32), pltpu.VMEM((GCH,), jnp.int32),
            pltpu.VMEM((GCH, D), jnp.float32), pltpu.VMEM((GCH, D), jnp.float32),
            pltpu.VMEM_SHARED((N_PAD, D), jnp.float32),
            pltpu.SemaphoreType.DMA, pltpu.SemaphoreType.DMA,
            pltpu.SemaphoreType.DMA, pltpu.SemaphoreType.DMA,
            pltpu.SemaphoreType.DMA, pltpu.SemaphoreType.DMA,
        ],
    )
    def gather_k(ztbl_hbm, idx_hbm, g_hbm,
                 idx_a, idx_b, rows_a, rows_b, spm_tbl,
                 sem_ia, sem_ib, sem_ga, sem_gb, sem_wa, sem_wb):
        c = lax.axis_index("c")
        s = lax.axis_index("s")
        rows_per_tile = N_PAD // NS
        my_rows = pl.ds(s * rows_per_tile, rows_per_tile)
        pltpu.sync_copy(ztbl_hbm.at[c].at[my_rows], spm_tbl.at[my_rows])
        plsc.subcore_barrier()

        base = s * EPT

        def off(j):
            return base + jnp.minimum(j * GCH, EPT - GCH)

        def fire_idx(j, ib, sem):
            return pltpu.async_copy(
                idx_hbm.at[pl.ds(c * E + e0 + off(j), GCH)], ib, sem)

        def fire_gather(ib, rb, sem):
            return pltpu.async_copy(spm_tbl.at[ib], rb, sem)

        def fire_write(j, rb, sem):
            return pltpu.async_copy(rb, g_hbm.at[c].at[pl.ds(off(j), GCH)],
                                    sem)

        def i_b_mk():
            return pltpu.make_async_copy(idx_hbm.at[pl.ds(0, GCH)],
                                         idx_b, sem_ib)

        def g_a_mk():
            return pltpu.make_async_copy(spm_tbl.at[idx_a], rows_a, sem_ga)

        fire_idx(0, idx_a, sem_ia).wait()
        fire_gather(idx_a, rows_a, sem_ga)
        fire_idx(1, idx_b, sem_ib)

        def body(k, carry):
            j0 = 2 * k
            j1 = j0 + 1
            j2 = j0 + 2
            j3 = j0 + 3
            i_b_mk().wait()
            g_b = fire_gather(idx_b, rows_b, sem_gb)
            g_a_mk().wait()
            w_a = fire_write(j0, rows_a, sem_wa)
            i_a = fire_idx(j2, idx_a, sem_ia)
            g_b.wait()
            w_a.wait()
            w_b = fire_write(j1, rows_b, sem_wb)
            i_a.wait()
            fire_gather(idx_a, rows_a, sem_ga)
            fire_idx(j3, idx_b, sem_ib)
            w_b.wait()
            return carry

        lax.fori_loop(0, NPAIR_G, body, 0)

        g_a_mk().wait()
        w_last = fire_write(NCHUNK_G - 1, rows_a, sem_wa)
        i_b_mk().wait()
        w_last.wait()

    return gather_k


def _make_scatter(e0):
    @functools.partial(
        pl.kernel,
        mesh=_mesh,
        out_type=jax.ShapeDtypeStruct((NC, N_PAD, D), jnp.float32),
        scratch_types=[
            pltpu.VMEM((SCH,), jnp.int32), pltpu.VMEM((SCH,), jnp.int32),
            pltpu.VMEM((SCH, D), jnp.float32), pltpu.VMEM((SCH, D), jnp.float32),
            pltpu.VMEM_SHARED((N_PAD, D), jnp.float32),
            pltpu.SemaphoreType.DMA, pltpu.SemaphoreType.DMA,
            pltpu.SemaphoreType.DMA, pltpu.SemaphoreType.DMA,
            pltpu.SemaphoreType.DMA,
        ],
    )
    def scatter_k(upd_hbm, r_hbm, zeros_hbm, agg_hbm,
                  idx_a, idx_b, rows_a, rows_b, acc_sh,
                  sem_ia, sem_ib, sem_la, sem_lb, sem_z):
        c = lax.axis_index("c")
        s = lax.axis_index("s")
        rows_per_tile = N_PAD // NS
        my_rows = pl.ds(s * rows_per_tile, rows_per_tile)
        zc = pltpu.async_copy(zeros_hbm.at[my_rows], acc_sh.at[my_rows],
                              sem_z)

        base = c * (EH // NC) + s * EPW

        def off(j):
            return base + jnp.minimum(j, NCHUNK_S - 1) * SCH

        def fire_idx(j, ib, sem):
            return pltpu.async_copy(r_hbm.at[pl.ds(e0 + off(j), SCH)], ib, sem)

        def fire_load(j, rb, sem):
            return pltpu.async_copy(upd_hbm.at[pl.ds(off(j), SCH)], rb, sem)

        def wait_ib():
            pltpu.make_async_copy(r_hbm.at[pl.ds(0, SCH)], idx_b, sem_ib).wait()

        def wait_lb():
            pltpu.make_async_copy(upd_hbm.at[pl.ds(0, SCH)], rows_b,
                                  sem_lb).wait()

        ia = fire_idx(0, idx_a, sem_ia)
        la = fire_load(0, rows_a, sem_la)
        fire_idx(1, idx_b, sem_ib)
        fire_load(1, rows_b, sem_lb)
        zc.wait()
        plsc.subcore_barrier()
        ia.wait()
        la.wait()

        def body(k, carry):
            j2 = 2 * k + 2
            j3 = 2 * k + 3
            pltpu.sync_copy(rows_a, acc_sh.at[idx_a], add=True)
            ia2 = fire_idx(j2, idx_a, sem_ia)
            la2 = fire_load(j2, rows_a, sem_la)
            wait_ib()
            wait_lb()
            pltpu.sync_copy(rows_b, acc_sh.at[idx_b], add=True)
            fire_idx(j3, idx_b, sem_ib)
            fire_load(j3, rows_b, sem_lb)
            ia2.wait()
            la2.wait()
            return carry

        lax.fori_loop(0, NPAIR_S, body, 0)

        pltpu.sync_copy(rows_a, acc_sh.at[idx_a], add=True)
        wait_ib()
        wait_lb()

        plsc.subcore_barrier()
        pltpu.sync_copy(acc_sh.at[my_rows], agg_hbm.at[c].at[my_rows])

    return scatter_k


_gathers = [_make_gather(q * EQ) for q in range(4)]
_scatter_h1 = _make_scatter(0)
_scatter_h2 = _make_scatter(EH)


def kernel(node_features, edge_features, senders, receivers,
           We1, be1, We2, be2, ln_e_scale, ln_e_bias,
           Wn1, bn1, Wn2, bn2, ln_n_scale, ln_n_bias):
    s32 = senders.astype(jnp.int32)
    r32 = receivers.astype(jnp.int32)

    ztbl = _proj(node_features, We1[:D], We1[D:2 * D])
    idx_cat = jnp.concatenate([s32, r32])
    w1c = We1[2 * D:]
    eb = (w1c, be1.reshape(1, D), We2, be2.reshape(1, D),
          ln_e_scale.reshape(1, D), ln_e_bias.reshape(1, D))

    gq = [g(ztbl, idx_cat) for g in _gathers]
    upd1a, new_v1 = _edge_mlp_q(0, None, None, gq[0], edge_features, *eb)
    upd1, new_v2 = _edge_mlp_q(1, new_v1, upd1a, gq[1], edge_features, *eb)
    upd2a, new_v3 = _edge_mlp_q(2, new_v2, None, gq[2], edge_features, *eb)
    upd2, new_edge = _edge_mlp_q(3, new_v3, upd2a, gq[3], edge_features, *eb)

    zeros = jnp.zeros((N_PAD, D), jnp.float32)
    agg_a = _scatter_h1(upd1, r32, zeros)
    agg_b = _scatter_h2(upd2, r32, zeros)

    new_node = _node_mlp(
        node_features, agg_a, agg_b,
        Wn1[:D], Wn1[D:], bn1.reshape(1, D), Wn2, bn2.reshape(1, D),
        ln_n_scale.reshape(1, D), ln_n_bias.reshape(1, D))
    return new_node, new_edge


# speedup vs baseline: 1.0087x; 1.0087x over previous
import functools

import jax
import jax.numpy as jnp
from jax import lax
from jax.experimental import pallas as pl
from jax.experimental.pallas import tpu as pltpu
from jax.experimental.pallas import tpu_sc as plsc

N = 10000
E = 320000
D = 128
N_PAD = 10240

NC, NS = 2, 16
NW = NC * NS
EH = E // 2
EPW = EH // NW

EPT = EH // NS
GCH = 104
NCHUNK_G = (EPT + GCH - 1) // GCH
NPAIR_G = (NCHUNK_G - 1) // 2

SCH = 40
NCHUNK_S = EPW // SCH
NPAIR_S = (NCHUNK_S - 1) // 2

NODE_BLK = 2000
EDGE_BLK = 2000
HBLKS = EH // EDGE_BLK

_mesh = plsc.VectorSubcoreMesh(core_axis_name="c", subcore_axis_name="s")


def _proj_body(nf_ref, wa_ref, wb_ref, z_ref):
    nf = nf_ref[...]
    z_ref[0] = jnp.dot(nf, wa_ref[...], preferred_element_type=jnp.float32)
    z_ref[1] = jnp.dot(nf, wb_ref[...], preferred_element_type=jnp.float32)


def _edge_math(gs, gr, ef, w1, b1, w2, b2, sc, bi):
    pre = (gs + gr + b1
           + jnp.dot(ef, w1, preferred_element_type=jnp.float32))
    h = jnp.maximum(pre, 0.0)
    y = jnp.dot(h, w2, preferred_element_type=jnp.float32) + b2
    mean = jnp.mean(y, axis=-1, keepdims=True)
    var = jnp.mean((y - mean) ** 2, axis=-1, keepdims=True)
    upd = (y - mean) * lax.rsqrt(var + 1e-5) * sc + bi
    return upd, upd + ef


def _edge_body1(gs_ref, gr_ref, ef_ref, w1_ref, b1_ref, w2_ref, b2_ref,
                sc_ref, bi_ref, upd_ref, new_ref):
    upd, new = _edge_math(gs_ref[0], gr_ref[0], ef_ref[...], w1_ref[...],
                          b1_ref[...], w2_ref[...], b2_ref[...], sc_ref[...],
                          bi_ref[...])
    upd_ref[...] = upd
    new_ref[...] = new


def _edge_body2(alias_ref, gs_ref, gr_ref, ef_ref, w1_ref, b1_ref, w2_ref,
                b2_ref, sc_ref, bi_ref, upd_ref, new_ref):
    del alias_ref
    upd, new = _edge_math(gs_ref[0], gr_ref[0], ef_ref[...], w1_ref[...],
                          b1_ref[...], w2_ref[...], b2_ref[...], sc_ref[...],
                          bi_ref[...])
    upd_ref[...] = upd
    new_ref[...] = new


def _node_body(nf_ref, a0_ref, a1_ref, a2_ref, a3_ref, w1a_ref, w1b_ref,
               b1_ref, w2_ref, b2_ref, sc_ref, bi_ref, out_ref):
    nf = nf_ref[...]
    agg = (a0_ref[0] + a1_ref[0]) + (a2_ref[0] + a3_ref[0])
    pre = (jnp.dot(nf, w1a_ref[...], preferred_element_type=jnp.float32)
           + jnp.dot(agg, w1b_ref[...], preferred_element_type=jnp.float32)
           + b1_ref[...])
    h = jnp.maximum(pre, 0.0)
    y = jnp.dot(h, w2_ref[...], preferred_element_type=jnp.float32) + b2_ref[...]
    mean = jnp.mean(y, axis=-1, keepdims=True)
    var = jnp.mean((y - mean) ** 2, axis=-1, keepdims=True)
    out_ref[...] = ((y - mean) * lax.rsqrt(var + 1e-5) * sc_ref[...]
                    + bi_ref[...] + nf)


def _row_spec(blk):
    return pl.BlockSpec((blk, D), lambda i: (i, 0))


def _full_spec(shape):
    return pl.BlockSpec(shape, lambda i: tuple(0 for _ in shape))


def _proj(nf, wa, wb):
    blk = 2048
    return pl.pallas_call(
        _proj_body,
        grid=(N_PAD // blk,),
        in_specs=[_row_spec(blk), _full_spec((D, D)), _full_spec((D, D))],
        out_specs=pl.BlockSpec((2, blk, D), lambda i: (0, i, 0)),
        out_shape=jax.ShapeDtypeStruct((2, N_PAD, D), jnp.float32),
    )(nf, wa, wb)


def _half_spec(half):
    blk0 = half * HBLKS
    return pl.BlockSpec((EDGE_BLK, D), lambda i: (blk0 + i, 0))


def _g_spec(plane):
    return pl.BlockSpec((1, EDGE_BLK, D), lambda i: (plane, i, 0))


def _edge_mlp1(gcat, ef, w1, b1, w2, b2, sc, bi):
    wspecs = [_full_spec((D, D)), _full_spec((1, D)), _full_spec((D, D)),
              _full_spec((1, D)), _full_spec((1, D)), _full_spec((1, D))]
    return pl.pallas_call(
        _edge_body1,
        grid=(HBLKS,),
        in_specs=[_g_spec(0), _g_spec(1), _half_spec(0)] + wspecs,
        out_specs=[_row_spec(EDGE_BLK), _half_spec(0)],
        out_shape=[jax.ShapeDtypeStruct((EH, D), jnp.float32),
                   jax.ShapeDtypeStruct((E, D), jnp.float32)],
    )(gcat, gcat, ef, w1, b1, w2, b2, sc, bi)


def _edge_mlp2(new_prev, gcat, ef, w1, b1, w2, b2, sc, bi):
    wspecs = [_full_spec((D, D)), _full_spec((1, D)), _full_spec((D, D)),
              _full_spec((1, D)), _full_spec((1, D)), _full_spec((1, D))]
    return pl.pallas_call(
        _edge_body2,
        grid=(HBLKS,),
        in_specs=[pl.BlockSpec((8, D), lambda i: (0, 0)),
                  _g_spec(0), _g_spec(1), _half_spec(1)] + wspecs,
        out_specs=[_row_spec(EDGE_BLK), _half_spec(1)],
        out_shape=[jax.ShapeDtypeStruct((EH, D), jnp.float32),
                   jax.ShapeDtypeStruct((E, D), jnp.float32)],
        input_output_aliases={0: 1},
    )(new_prev, gcat, gcat, ef, w1, b1, w2, b2, sc, bi)


def _node_mlp(nf, agg_a, agg_b, w1a, w1b, b1, w2, b2, sc, bi):
    grid = N // NODE_BLK
    aspec = [pl.BlockSpec((1, NODE_BLK, D), lambda i: (0, i, 0)),
             pl.BlockSpec((1, NODE_BLK, D), lambda i: (1, i, 0))]
    return pl.pallas_call(
        _node_body,
        grid=(grid,),
        in_specs=[_row_spec(NODE_BLK)] + aspec + aspec
        + [_full_spec((D, D)), _full_spec((D, D)), _full_spec((1, D)),
           _full_spec((D, D)), _full_spec((1, D)), _full_spec((1, D)),
           _full_spec((1, D))],
        out_specs=_row_spec(NODE_BLK),
        out_shape=jax.ShapeDtypeStruct((N, D), jnp.float32),
    )(nf, agg_a, agg_a, agg_b, agg_b, w1a, w1b, b1, w2, b2, sc, bi)


def _make_gather(e0):
    @functools.partial(
        pl.kernel,
        mesh=_mesh,
        out_type=jax.ShapeDtypeStruct((2, EH, D), jnp.float32),
        scratch_types=[
            pltpu.VMEM((GCH,), jnp.int32), pltpu.VMEM((GCH,), jnp.int32),
            pltpu.VMEM((GCH, D), jnp.float32), pltpu.VMEM((GCH, D), jnp.float32),
            pltpu.VMEM_SHARED((N_PAD, D), jnp.float32),
            pltpu.SemaphoreType.DMA, pltpu.SemaphoreType.DMA,
            pltpu.SemaphoreType.DMA, pltpu.SemaphoreType.DMA,
            pltpu.SemaphoreType.DMA, pltpu.SemaphoreType.DMA,
        ],
    )
    def gather_k(ztbl_hbm, idx_hbm, g_hbm,
                 idx_a, idx_b, rows_a, rows_b, spm_tbl,
                 sem_ia, sem_ib, sem_ga, sem_gb, sem_wa, sem_wb):
        c = lax.axis_index("c")
        s = lax.axis_index("s")
        rows_per_tile = N_PAD // NS
        my_rows = pl.ds(s * rows_per_tile, rows_per_tile)
        pltpu.sync_copy(ztbl_hbm.at[c].at[my_rows], spm_tbl.at[my_rows])
        plsc.subcore_barrier()

        base = s * EPT

        def off(j):
            return base + jnp.minimum(j * GCH, EPT - GCH)

        def fire_idx(j, ib, sem):
            return pltpu.async_copy(
                idx_hbm.at[pl.ds(c * E + e0 + off(j), GCH)], ib, sem)

        def fire_gather(ib, rb, sem):
            return pltpu.async_copy(spm_tbl.at[ib], rb, sem)

        def fire_write(j, rb, sem):
            return pltpu.async_copy(rb, g_hbm.at[c].at[pl.ds(off(j), GCH)],
                                    sem)

        def i_b_mk():
            return pltpu.make_async_copy(idx_hbm.at[pl.ds(0, GCH)],
                                         idx_b, sem_ib)

        def g_a_mk():
            return pltpu.make_async_copy(spm_tbl.at[idx_a], rows_a, sem_ga)

        fire_idx(0, idx_a, sem_ia).wait()
        fire_gather(idx_a, rows_a, sem_ga)
        fire_idx(1, idx_b, sem_ib)

        def body(k, carry):
            j0 = 2 * k
            j1 = j0 + 1
            j2 = j0 + 2
            j3 = j0 + 3
            i_b_mk().wait()
            g_b = fire_gather(idx_b, rows_b, sem_gb)
            g_a_mk().wait()
            w_a = fire_write(j0, rows_a, sem_wa)
            i_a = fire_idx(j2, idx_a, sem_ia)
            g_b.wait()
            w_a.wait()
            w_b = fire_write(j1, rows_b, sem_wb)
            i_a.wait()
            fire_gather(idx_a, rows_a, sem_ga)
            fire_idx(j3, idx_b, sem_ib)
            w_b.wait()
            return carry

        lax.fori_loop(0, NPAIR_G, body, 0)

        g_a_mk().wait()
        w_last = fire_write(NCHUNK_G - 1, rows_a, sem_wa)
        i_b_mk().wait()
        w_last.wait()

    return gather_k


def _make_scatter(e0):
    @functools.partial(
        pl.kernel,
        mesh=_mesh,
        out_type=jax.ShapeDtypeStruct((NC, N_PAD, D), jnp.float32),
        scratch_types=[
            pltpu.VMEM((SCH,), jnp.int32), pltpu.VMEM((SCH,), jnp.int32),
            pltpu.VMEM((SCH, D), jnp.float32), pltpu.VMEM((SCH, D), jnp.float32),
            pltpu.VMEM_SHARED((N_PAD, D), jnp.float32),
            pltpu.SemaphoreType.DMA, pltpu.SemaphoreType.DMA,
            pltpu.SemaphoreType.DMA, pltpu.SemaphoreType.DMA,
            pltpu.SemaphoreType.DMA,
        ],
    )
    def scatter_k(upd_hbm, r_hbm, zeros_hbm, agg_hbm,
                  idx_a, idx_b, rows_a, rows_b, acc_sh,
                  sem_ia, sem_ib, sem_la, sem_lb, sem_z):
        c = lax.axis_index("c")
        s = lax.axis_index("s")
        rows_per_tile = N_PAD // NS
        my_rows = pl.ds(s * rows_per_tile, rows_per_tile)
        zc = pltpu.async_copy(zeros_hbm.at[my_rows], acc_sh.at[my_rows],
                              sem_z)

        base = c * (EH // NC) + s * EPW

        def off(j):
            return base + jnp.minimum(j, NCHUNK_S - 1) * SCH

        def fire_idx(j, ib, sem):
            return pltpu.async_copy(r_hbm.at[pl.ds(e0 + off(j), SCH)], ib, sem)

        def fire_load(j, rb, sem):
            return pltpu.async_copy(upd_hbm.at[pl.ds(off(j), SCH)], rb, sem)

        def wait_ib():
            pltpu.make_async_copy(r_hbm.at[pl.ds(0, SCH)], idx_b, sem_ib).wait()

        def wait_lb():
            pltpu.make_async_copy(upd_hbm.at[pl.ds(0, SCH)], rows_b,
                                  sem_lb).wait()

        ia = fire_idx(0, idx_a, sem_ia)
        la = fire_load(0, rows_a, sem_la)
        fire_idx(1, idx_b, sem_ib)
        fire_load(1, rows_b, sem_lb)
        zc.wait()
        plsc.subcore_barrier()
        ia.wait()
        la.wait()

        def body(k, carry):
            j2 = 2 * k + 2
            j3 = 2 * k + 3
            pltpu.sync_copy(rows_a, acc_sh.at[idx_a], add=True)
            ia2 = fire_idx(j2, idx_a, sem_ia)
            la2 = fire_load(j2, rows_a, sem_la)
            wait_ib()
            wait_lb()
            pltpu.sync_copy(rows_b, acc_sh.at[idx_b], add=True)
            fire_idx(j3, idx_b, sem_ib)
            fire_load(j3, rows_b, sem_lb)
            ia2.wait()
            la2.wait()
            return carry

        lax.fori_loop(0, NPAIR_S, body, 0)

        pltpu.sync_copy(rows_a, acc_sh.at[idx_a], add=True)
        wait_ib()
        wait_lb()

        plsc.subcore_barrier()
        pltpu.sync_copy(acc_sh.at[my_rows], agg_hbm.at[c].at[my_rows])

    return scatter_k


_gather_h1 = _make_gather(0)
_gather_h2 = _make_gather(EH)
_scatter_h1 = _make_scatter(0)
_scatter_h2 = _make_scatter(EH)


def kernel(node_features, edge_features, senders, receivers,
           We1, be1, We2, be2, ln_e_scale, ln_e_bias,
           Wn1, bn1, Wn2, bn2, ln_n_scale, ln_n_bias):
    s32 = senders.astype(jnp.int32)
    r32 = receivers.astype(jnp.int32)

    ztbl = _proj(node_features, We1[:D], We1[D:2 * D])
    idx_cat = jnp.concatenate([s32, r32])
    w1c = We1[2 * D:]
    eb = (w1c, be1.reshape(1, D), We2, be2.reshape(1, D),
          ln_e_scale.reshape(1, D), ln_e_bias.reshape(1, D))

    g1 = _gather_h1(ztbl, idx_cat)
    g2 = _gather_h2(ztbl, idx_cat)
    upd1, new_v1 = _edge_mlp1(g1, edge_features, *eb)
    upd2, new_edge = _edge_mlp2(new_v1, g2, edge_features, *eb)

    zeros = jnp.zeros((N_PAD, D), jnp.float32)
    agg_a = _scatter_h1(upd1, r32, zeros)
    agg_b = _scatter_h2(upd2, r32, zeros)

    new_node = _node_mlp(
        node_features, agg_a, agg_b,
        Wn1[:D], Wn1[D:], bn1.reshape(1, D), Wn2, bn2.reshape(1, D),
        ln_n_scale.reshape(1, D), ln_n_bias.reshape(1, D))
    return new_node, new_edge


# speedup vs baseline: 1.0376x; 1.0287x over previous
import functools

import jax
import jax.numpy as jnp
from jax import lax
from jax.experimental import pallas as pl
from jax.experimental.pallas import tpu as pltpu
from jax.experimental.pallas import tpu_sc as plsc

N = 10000
E = 320000
D = 128
N_PAD = 10240

NC, NS = 2, 16
NW = NC * NS
EH = E // 2
EPW = EH // NW

EPT = EH // NS
GCH = 104
NCHUNK_G = (EPT + GCH - 1) // GCH
NPAIR_G = (NCHUNK_G - 1) // 2

SCH = 40
NCHUNK_S = EPW // SCH
NPAIR_S = (NCHUNK_S - 1) // 2

NODE_BLK = 2000
EDGE_BLK = 4000
HBLKS = EH // EDGE_BLK

_mesh = plsc.VectorSubcoreMesh(core_axis_name="c", subcore_axis_name="s")


def _proj_body(nf_ref, wa_ref, wb_ref, z_ref):
    nf = nf_ref[...]
    z_ref[0] = jnp.dot(nf, wa_ref[...], preferred_element_type=jnp.float32)
    z_ref[1] = jnp.dot(nf, wb_ref[...], preferred_element_type=jnp.float32)


def _edge_math(gs, gr, ef, w1, b1, w2, b2, sc, bi):
    pre = (gs + gr + b1
           + jnp.dot(ef, w1, preferred_element_type=jnp.float32))
    h = jnp.maximum(pre, 0.0)
    y = jnp.dot(h, w2, preferred_element_type=jnp.float32) + b2
    mean = jnp.mean(y, axis=-1, keepdims=True)
    var = jnp.mean((y - mean) ** 2, axis=-1, keepdims=True)
    upd = (y - mean) * lax.rsqrt(var + 1e-5) * sc + bi
    return upd, upd + ef


def _edge_body1(gs_ref, gr_ref, ef_ref, w1_ref, b1_ref, w2_ref, b2_ref,
                sc_ref, bi_ref, upd_ref, new_ref):
    upd, new = _edge_math(gs_ref[0], gr_ref[0], ef_ref[...], w1_ref[...],
                          b1_ref[...], w2_ref[...], b2_ref[...], sc_ref[...],
                          bi_ref[...])
    upd_ref[...] = upd
    new_ref[...] = new


def _edge_body2(alias_ref, gs_ref, gr_ref, ef_ref, w1_ref, b1_ref, w2_ref,
                b2_ref, sc_ref, bi_ref, upd_ref, new_ref):
    del alias_ref
    upd, new = _edge_math(gs_ref[0], gr_ref[0], ef_ref[...], w1_ref[...],
                          b1_ref[...], w2_ref[...], b2_ref[...], sc_ref[...],
                          bi_ref[...])
    upd_ref[...] = upd
    new_ref[...] = new


def _node_body(nf_ref, a0_ref, a1_ref, a2_ref, a3_ref, w1a_ref, w1b_ref,
               b1_ref, w2_ref, b2_ref, sc_ref, bi_ref, out_ref):
    nf = nf_ref[...]
    agg = (a0_ref[0] + a1_ref[0]) + (a2_ref[0] + a3_ref[0])
    pre = (jnp.dot(nf, w1a_ref[...], preferred_element_type=jnp.float32)
           + jnp.dot(agg, w1b_ref[...], preferred_element_type=jnp.float32)
           + b1_ref[...])
    h = jnp.maximum(pre, 0.0)
    y = jnp.dot(h, w2_ref[...], preferred_element_type=jnp.float32) + b2_ref[...]
    mean = jnp.mean(y, axis=-1, keepdims=True)
    var = jnp.mean((y - mean) ** 2, axis=-1, keepdims=True)
    out_ref[...] = ((y - mean) * lax.rsqrt(var + 1e-5) * sc_ref[...]
                    + bi_ref[...] + nf)


def _row_spec(blk):
    return pl.BlockSpec((blk, D), lambda i: (i, 0))


def _full_spec(shape):
    return pl.BlockSpec(shape, lambda i: tuple(0 for _ in shape))


def _proj(nf, wa, wb):
    blk = 2048
    return pl.pallas_call(
        _proj_body,
        grid=(N_PAD // blk,),
        in_specs=[_row_spec(blk), _full_spec((D, D)), _full_spec((D, D))],
        out_specs=pl.BlockSpec((2, blk, D), lambda i: (0, i, 0)),
        out_shape=jax.ShapeDtypeStruct((2, N_PAD, D), jnp.float32),
    )(nf, wa, wb)


def _half_spec(half):
    blk0 = half * HBLKS
    return pl.BlockSpec((EDGE_BLK, D), lambda i: (blk0 + i, 0))


def _g_spec(plane):
    return pl.BlockSpec((1, EDGE_BLK, D), lambda i: (plane, i, 0))


def _edge_mlp1(gcat, ef, w1, b1, w2, b2, sc, bi):
    wspecs = [_full_spec((D, D)), _full_spec((1, D)), _full_spec((D, D)),
              _full_spec((1, D)), _full_spec((1, D)), _full_spec((1, D))]
    return pl.pallas_call(
        _edge_body1,
        grid=(HBLKS,),
        in_specs=[_g_spec(0), _g_spec(1), _half_spec(0)] + wspecs,
        out_specs=[_row_spec(EDGE_BLK), _half_spec(0)],
        out_shape=[jax.ShapeDtypeStruct((EH, D), jnp.float32),
                   jax.ShapeDtypeStruct((E, D), jnp.float32)],
    )(gcat, gcat, ef, w1, b1, w2, b2, sc, bi)


def _edge_mlp2(new_prev, gcat, ef, w1, b1, w2, b2, sc, bi):
    wspecs = [_full_spec((D, D)), _full_spec((1, D)), _full_spec((D, D)),
              _full_spec((1, D)), _full_spec((1, D)), _full_spec((1, D))]
    return pl.pallas_call(
        _edge_body2,
        grid=(HBLKS,),
        in_specs=[pl.BlockSpec((8, D), lambda i: (0, 0)),
                  _g_spec(0), _g_spec(1), _half_spec(1)] + wspecs,
        out_specs=[_row_spec(EDGE_BLK), _half_spec(1)],
        out_shape=[jax.ShapeDtypeStruct((EH, D), jnp.float32),
                   jax.ShapeDtypeStruct((E, D), jnp.float32)],
        input_output_aliases={0: 1},
    )(new_prev, gcat, gcat, ef, w1, b1, w2, b2, sc, bi)


def _node_mlp(nf, agg_a, agg_b, w1a, w1b, b1, w2, b2, sc, bi):
    grid = N // NODE_BLK
    aspec = [pl.BlockSpec((1, NODE_BLK, D), lambda i: (0, i, 0)),
             pl.BlockSpec((1, NODE_BLK, D), lambda i: (1, i, 0))]
    return pl.pallas_call(
        _node_body,
        grid=(grid,),
        in_specs=[_row_spec(NODE_BLK)] + aspec + aspec
        + [_full_spec((D, D)), _full_spec((D, D)), _full_spec((1, D)),
           _full_spec((D, D)), _full_spec((1, D)), _full_spec((1, D)),
           _full_spec((1, D))],
        out_specs=_row_spec(NODE_BLK),
        out_shape=jax.ShapeDtypeStruct((N, D), jnp.float32),
    )(nf, agg_a, agg_a, agg_b, agg_b, w1a, w1b, b1, w2, b2, sc, bi)


def _make_gather(e0):
    @functools.partial(
        pl.kernel,
        mesh=_mesh,
        out_type=jax.ShapeDtypeStruct((2, EH, D), jnp.float32),
        scratch_types=[
            pltpu.VMEM((GCH,), jnp.int32), pltpu.VMEM((GCH,), jnp.int32),
            pltpu.VMEM((GCH, D), jnp.float32), pltpu.VMEM((GCH, D), jnp.float32),
            pltpu.VMEM_SHARED((N_PAD, D), jnp.float32),
            pltpu.SemaphoreType.DMA, pltpu.SemaphoreType.DMA,
            pltpu.SemaphoreType.DMA, pltpu.SemaphoreType.DMA,
            pltpu.SemaphoreType.DMA, pltpu.SemaphoreType.DMA,
        ],
    )
    def gather_k(ztbl_hbm, idx_hbm, g_hbm,
                 idx_a, idx_b, rows_a, rows_b, spm_tbl,
                 sem_ia, sem_ib, sem_ga, sem_gb, sem_wa, sem_wb):
        c = lax.axis_index("c")
        s = lax.axis_index("s")
        rows_per_tile = N_PAD // NS
        my_rows = pl.ds(s * rows_per_tile, rows_per_tile)
        pltpu.sync_copy(ztbl_hbm.at[c].at[my_rows], spm_tbl.at[my_rows])
        plsc.subcore_barrier()

        base = s * EPT

        def off(j):
            return base + jnp.minimum(j * GCH, EPT - GCH)

        def fire_idx(j, ib, sem):
            return pltpu.async_copy(
                idx_hbm.at[pl.ds(c * E + e0 + off(j), GCH)], ib, sem)

        def fire_gather(ib, rb, sem):
            return pltpu.async_copy(spm_tbl.at[ib], rb, sem)

        def fire_write(j, rb, sem):
            return pltpu.async_copy(rb, g_hbm.at[c].at[pl.ds(off(j), GCH)],
                                    sem)

        def i_b_mk():
            return pltpu.make_async_copy(idx_hbm.at[pl.ds(0, GCH)],
                                         idx_b, sem_ib)

        def g_a_mk():
            return pltpu.make_async_copy(spm_tbl.at[idx_a], rows_a, sem_ga)

        fire_idx(0, idx_a, sem_ia).wait()
        fire_gather(idx_a, rows_a, sem_ga)
        fire_idx(1, idx_b, sem_ib)

        def body(k, carry):
            j0 = 2 * k
            j1 = j0 + 1
            j2 = j0 + 2
            j3 = j0 + 3
            i_b_mk().wait()
            g_b = fire_gather(idx_b, rows_b, sem_gb)
            g_a_mk().wait()
            w_a = fire_write(j0, rows_a, sem_wa)
            i_a = fire_idx(j2, idx_a, sem_ia)
            g_b.wait()
            w_a.wait()
            w_b = fire_write(j1, rows_b, sem_wb)
            i_a.wait()
            fire_gather(idx_a, rows_a, sem_ga)
            fire_idx(j3, idx_b, sem_ib)
            w_b.wait()
            return carry

        lax.fori_loop(0, NPAIR_G, body, 0)

        g_a_mk().wait()
        w_last = fire_write(NCHUNK_G - 1, rows_a, sem_wa)
        i_b_mk().wait()
        w_last.wait()

    return gather_k


def _make_scatter(e0):
    @functools.partial(
        pl.kernel,
        mesh=_mesh,
        out_type=jax.ShapeDtypeStruct((NC, N_PAD, D), jnp.float32),
        scratch_types=[
            pltpu.VMEM((SCH,), jnp.int32), pltpu.VMEM((SCH,), jnp.int32),
            pltpu.VMEM((SCH, D), jnp.float32), pltpu.VMEM((SCH, D), jnp.float32),
            pltpu.VMEM_SHARED((N_PAD, D), jnp.float32),
            pltpu.SemaphoreType.DMA, pltpu.SemaphoreType.DMA,
            pltpu.SemaphoreType.DMA, pltpu.SemaphoreType.DMA,
            pltpu.SemaphoreType.DMA,
        ],
    )
    def scatter_k(upd_hbm, r_hbm, zeros_hbm, agg_hbm,
                  idx_a, idx_b, rows_a, rows_b, acc_sh,
                  sem_ia, sem_ib, sem_la, sem_lb, sem_z):
        c = lax.axis_index("c")
        s = lax.axis_index("s")
        rows_per_tile = N_PAD // NS
        my_rows = pl.ds(s * rows_per_tile, rows_per_tile)
        zc = pltpu.async_copy(zeros_hbm.at[my_rows], acc_sh.at[my_rows],
                              sem_z)

        base = c * (EH // NC) + s * EPW

        def off(j):
            return base + jnp.minimum(j, NCHUNK_S - 1) * SCH

        def fire_idx(j, ib, sem):
            return pltpu.async_copy(r_hbm.at[pl.ds(e0 + off(j), SCH)], ib, sem)

        def fire_load(j, rb, sem):
            return pltpu.async_copy(upd_hbm.at[pl.ds(off(j), SCH)], rb, sem)

        def wait_ib():
            pltpu.make_async_copy(r_hbm.at[pl.ds(0, SCH)], idx_b, sem_ib).wait()

        def wait_lb():
            pltpu.make_async_copy(upd_hbm.at[pl.ds(0, SCH)], rows_b,
                                  sem_lb).wait()

        ia = fire_idx(0, idx_a, sem_ia)
        la = fire_load(0, rows_a, sem_la)
        fire_idx(1, idx_b, sem_ib)
        fire_load(1, rows_b, sem_lb)
        zc.wait()
        plsc.subcore_barrier()
        ia.wait()
        la.wait()

        def body(k, carry):
            j2 = 2 * k + 2
            j3 = 2 * k + 3
            pltpu.sync_copy(rows_a, acc_sh.at[idx_a], add=True)
            ia2 = fire_idx(j2, idx_a, sem_ia)
            la2 = fire_load(j2, rows_a, sem_la)
            wait_ib()
            wait_lb()
            pltpu.sync_copy(rows_b, acc_sh.at[idx_b], add=True)
            fire_idx(j3, idx_b, sem_ib)
            fire_load(j3, rows_b, sem_lb)
            ia2.wait()
            la2.wait()
            return carry

        lax.fori_loop(0, NPAIR_S, body, 0)

        pltpu.sync_copy(rows_a, acc_sh.at[idx_a], add=True)
        wait_ib()
        wait_lb()

        plsc.subcore_barrier()
        pltpu.sync_copy(acc_sh.at[my_rows], agg_hbm.at[c].at[my_rows])

    return scatter_k


_gather_h1 = _make_gather(0)
_gather_h2 = _make_gather(EH)
_scatter_h1 = _make_scatter(0)
_scatter_h2 = _make_scatter(EH)


def kernel(node_features, edge_features, senders, receivers,
           We1, be1, We2, be2, ln_e_scale, ln_e_bias,
           Wn1, bn1, Wn2, bn2, ln_n_scale, ln_n_bias):
    s32 = senders.astype(jnp.int32)
    r32 = receivers.astype(jnp.int32)

    ztbl = _proj(node_features, We1[:D], We1[D:2 * D])
    idx_cat = jnp.concatenate([s32, r32])
    w1c = We1[2 * D:]
    eb = (w1c, be1.reshape(1, D), We2, be2.reshape(1, D),
          ln_e_scale.reshape(1, D), ln_e_bias.reshape(1, D))

    g1 = _gather_h1(ztbl, idx_cat)
    g2 = _gather_h2(ztbl, idx_cat)
    upd1, new_v1 = _edge_mlp1(g1, edge_features, *eb)
    upd2, new_edge = _edge_mlp2(new_v1, g2, edge_features, *eb)

    zeros = jnp.zeros((N_PAD, D), jnp.float32)
    agg_a = _scatter_h1(upd1, r32, zeros)
    agg_b = _scatter_h2(upd2, r32, zeros)

    new_node = _node_mlp(
        node_features, agg_a, agg_b,
        Wn1[:D], Wn1[D:], bn1.reshape(1, D), Wn2, bn2.reshape(1, D),
        ln_n_scale.reshape(1, D), ln_n_bias.reshape(1, D))
    return new_node, new_edge


# speedup vs baseline: 1.0394x; 1.0017x over previous
import functools

import jax
import jax.numpy as jnp
from jax import lax
from jax.experimental import pallas as pl
from jax.experimental.pallas import tpu as pltpu
from jax.experimental.pallas import tpu_sc as plsc

N = 10000
E = 320000
D = 128
N_PAD = 10240

NC, NS = 2, 16
NW = NC * NS
EH = E // 2
EPW = EH // NW

EPT = EH // NS
GCH = 104
NCHUNK_G = (EPT + GCH - 1) // GCH
NPAIR_G = (NCHUNK_G - 1) // 2

SCH = 40
NCHUNK_S = EPW // SCH
NPAIR_S = (NCHUNK_S - 1) // 2

NODE_BLK = 2000
EDGE_BLK = 8000
HBLKS = EH // EDGE_BLK

_mesh = plsc.VectorSubcoreMesh(core_axis_name="c", subcore_axis_name="s")


def _proj_body(nf_ref, wa_ref, wb_ref, z_ref):
    nf = nf_ref[...]
    z_ref[0] = jnp.dot(nf, wa_ref[...], preferred_element_type=jnp.float32)
    z_ref[1] = jnp.dot(nf, wb_ref[...], preferred_element_type=jnp.float32)


def _edge_math(gs, gr, ef, w1, b1, w2, b2, sc, bi):
    pre = (gs + gr + b1
           + jnp.dot(ef, w1, preferred_element_type=jnp.float32))
    h = jnp.maximum(pre, 0.0)
    y = jnp.dot(h, w2, preferred_element_type=jnp.float32) + b2
    mean = jnp.mean(y, axis=-1, keepdims=True)
    var = jnp.mean((y - mean) ** 2, axis=-1, keepdims=True)
    upd = (y - mean) * lax.rsqrt(var + 1e-5) * sc + bi
    return upd, upd + ef


def _edge_body1(gs_ref, gr_ref, ef_ref, w1_ref, b1_ref, w2_ref, b2_ref,
                sc_ref, bi_ref, upd_ref, new_ref):
    upd, new = _edge_math(gs_ref[0], gr_ref[0], ef_ref[...], w1_ref[...],
                          b1_ref[...], w2_ref[...], b2_ref[...], sc_ref[...],
                          bi_ref[...])
    upd_ref[...] = upd
    new_ref[...] = new


def _edge_body2(alias_ref, gs_ref, gr_ref, ef_ref, w1_ref, b1_ref, w2_ref,
                b2_ref, sc_ref, bi_ref, upd_ref, new_ref):
    del alias_ref
    upd, new = _edge_math(gs_ref[0], gr_ref[0], ef_ref[...], w1_ref[...],
                          b1_ref[...], w2_ref[...], b2_ref[...], sc_ref[...],
                          bi_ref[...])
    upd_ref[...] = upd
    new_ref[...] = new


def _node_body(nf_ref, a0_ref, a1_ref, a2_ref, a3_ref, w1a_ref, w1b_ref,
               b1_ref, w2_ref, b2_ref, sc_ref, bi_ref, out_ref):
    nf = nf_ref[...]
    agg = (a0_ref[0] + a1_ref[0]) + (a2_ref[0] + a3_ref[0])
    pre = (jnp.dot(nf, w1a_ref[...], preferred_element_type=jnp.float32)
           + jnp.dot(agg, w1b_ref[...], preferred_element_type=jnp.float32)
           + b1_ref[...])
    h = jnp.maximum(pre, 0.0)
    y = jnp.dot(h, w2_ref[...], preferred_element_type=jnp.float32) + b2_ref[...]
    mean = jnp.mean(y, axis=-1, keepdims=True)
    var = jnp.mean((y - mean) ** 2, axis=-1, keepdims=True)
    out_ref[...] = ((y - mean) * lax.rsqrt(var + 1e-5) * sc_ref[...]
                    + bi_ref[...] + nf)


def _row_spec(blk):
    return pl.BlockSpec((blk, D), lambda i: (i, 0))


def _full_spec(shape):
    return pl.BlockSpec(shape, lambda i: tuple(0 for _ in shape))


def _proj(nf, wa, wb):
    blk = 2048
    return pl.pallas_call(
        _proj_body,
        grid=(N_PAD // blk,),
        in_specs=[_row_spec(blk), _full_spec((D, D)), _full_spec((D, D))],
        out_specs=pl.BlockSpec((2, blk, D), lambda i: (0, i, 0)),
        out_shape=jax.ShapeDtypeStruct((2, N_PAD, D), jnp.float32),
    )(nf, wa, wb)


def _half_spec(half):
    blk0 = half * HBLKS
    return pl.BlockSpec((EDGE_BLK, D), lambda i: (blk0 + i, 0))


def _g_spec(plane):
    return pl.BlockSpec((1, EDGE_BLK, D), lambda i: (plane, i, 0))


def _edge_mlp1(gcat, ef, w1, b1, w2, b2, sc, bi):
    wspecs = [_full_spec((D, D)), _full_spec((1, D)), _full_spec((D, D)),
              _full_spec((1, D)), _full_spec((1, D)), _full_spec((1, D))]
    return pl.pallas_call(
        _edge_body1,
        grid=(HBLKS,),
        in_specs=[_g_spec(0), _g_spec(1), _half_spec(0)] + wspecs,
        out_specs=[_row_spec(EDGE_BLK), _half_spec(0)],
        out_shape=[jax.ShapeDtypeStruct((EH, D), jnp.float32),
                   jax.ShapeDtypeStruct((E, D), jnp.float32)],
    )(gcat, gcat, ef, w1, b1, w2, b2, sc, bi)


def _edge_mlp2(new_prev, gcat, ef, w1, b1, w2, b2, sc, bi):
    wspecs = [_full_spec((D, D)), _full_spec((1, D)), _full_spec((D, D)),
              _full_spec((1, D)), _full_spec((1, D)), _full_spec((1, D))]
    return pl.pallas_call(
        _edge_body2,
        grid=(HBLKS,),
        in_specs=[pl.BlockSpec((8, D), lambda i: (0, 0)),
                  _g_spec(0), _g_spec(1), _half_spec(1)] + wspecs,
        out_specs=[_row_spec(EDGE_BLK), _half_spec(1)],
        out_shape=[jax.ShapeDtypeStruct((EH, D), jnp.float32),
                   jax.ShapeDtypeStruct((E, D), jnp.float32)],
        input_output_aliases={0: 1},
    )(new_prev, gcat, gcat, ef, w1, b1, w2, b2, sc, bi)


def _node_mlp(nf, agg_a, agg_b, w1a, w1b, b1, w2, b2, sc, bi):
    grid = N // NODE_BLK
    aspec = [pl.BlockSpec((1, NODE_BLK, D), lambda i: (0, i, 0)),
             pl.BlockSpec((1, NODE_BLK, D), lambda i: (1, i, 0))]
    return pl.pallas_call(
        _node_body,
        grid=(grid,),
        in_specs=[_row_spec(NODE_BLK)] + aspec + aspec
        + [_full_spec((D, D)), _full_spec((D, D)), _full_spec((1, D)),
           _full_spec((D, D)), _full_spec((1, D)), _full_spec((1, D)),
           _full_spec((1, D))],
        out_specs=_row_spec(NODE_BLK),
        out_shape=jax.ShapeDtypeStruct((N, D), jnp.float32),
    )(nf, agg_a, agg_a, agg_b, agg_b, w1a, w1b, b1, w2, b2, sc, bi)


def _make_gather(e0):
    @functools.partial(
        pl.kernel,
        mesh=_mesh,
        out_type=jax.ShapeDtypeStruct((2, EH, D), jnp.float32),
        scratch_types=[
            pltpu.VMEM((GCH,), jnp.int32), pltpu.VMEM((GCH,), jnp.int32),
            pltpu.VMEM((GCH, D), jnp.float32), pltpu.VMEM((GCH, D), jnp.float32),
            pltpu.VMEM_SHARED((N_PAD, D), jnp.float32),
            pltpu.SemaphoreType.DMA, pltpu.SemaphoreType.DMA,
            pltpu.SemaphoreType.DMA, pltpu.SemaphoreType.DMA,
            pltpu.SemaphoreType.DMA, pltpu.SemaphoreType.DMA,
        ],
    )
    def gather_k(ztbl_hbm, idx_hbm, g_hbm,
                 idx_a, idx_b, rows_a, rows_b, spm_tbl,
                 sem_ia, sem_ib, sem_ga, sem_gb, sem_wa, sem_wb):
        c = lax.axis_index("c")
        s = lax.axis_index("s")
        rows_per_tile = N_PAD // NS
        my_rows = pl.ds(s * rows_per_tile, rows_per_tile)
        pltpu.sync_copy(ztbl_hbm.at[c].at[my_rows], spm_tbl.at[my_rows])
        plsc.subcore_barrier()

        base = s * EPT

        def off(j):
            return base + jnp.minimum(j * GCH, EPT - GCH)

        def fire_idx(j, ib, sem):
            return pltpu.async_copy(
                idx_hbm.at[pl.ds(c * E + e0 + off(j), GCH)], ib, sem)

        def fire_gather(ib, rb, sem):
            return pltpu.async_copy(spm_tbl.at[ib], rb, sem)

        def fire_write(j, rb, sem):
            return pltpu.async_copy(rb, g_hbm.at[c].at[pl.ds(off(j), GCH)],
                                    sem)

        def i_b_mk():
            return pltpu.make_async_copy(idx_hbm.at[pl.ds(0, GCH)],
                                         idx_b, sem_ib)

        def g_a_mk():
            return pltpu.make_async_copy(spm_tbl.at[idx_a], rows_a, sem_ga)

        fire_idx(0, idx_a, sem_ia).wait()
        fire_gather(idx_a, rows_a, sem_ga)
        fire_idx(1, idx_b, sem_ib)

        def body(k, carry):
            j0 = 2 * k
            j1 = j0 + 1
            j2 = j0 + 2
            j3 = j0 + 3
            i_b_mk().wait()
            g_b = fire_gather(idx_b, rows_b, sem_gb)
            g_a_mk().wait()
            w_a = fire_write(j0, rows_a, sem_wa)
            i_a = fire_idx(j2, idx_a, sem_ia)
            g_b.wait()
            w_a.wait()
            w_b = fire_write(j1, rows_b, sem_wb)
            i_a.wait()
            fire_gather(idx_a, rows_a, sem_ga)
            fire_idx(j3, idx_b, sem_ib)
            w_b.wait()
            return carry

        lax.fori_loop(0, NPAIR_G, body, 0)

        g_a_mk().wait()
        w_last = fire_write(NCHUNK_G - 1, rows_a, sem_wa)
        i_b_mk().wait()
        w_last.wait()

    return gather_k


def _make_scatter(e0):
    @functools.partial(
        pl.kernel,
        mesh=_mesh,
        out_type=jax.ShapeDtypeStruct((NC, N_PAD, D), jnp.float32),
        scratch_types=[
            pltpu.VMEM((SCH,), jnp.int32), pltpu.VMEM((SCH,), jnp.int32),
            pltpu.VMEM((SCH, D), jnp.float32), pltpu.VMEM((SCH, D), jnp.float32),
            pltpu.VMEM_SHARED((N_PAD, D), jnp.float32),
            pltpu.SemaphoreType.DMA, pltpu.SemaphoreType.DMA,
            pltpu.SemaphoreType.DMA, pltpu.SemaphoreType.DMA,
            pltpu.SemaphoreType.DMA,
        ],
    )
    def scatter_k(upd_hbm, r_hbm, zeros_hbm, agg_hbm,
                  idx_a, idx_b, rows_a, rows_b, acc_sh,
                  sem_ia, sem_ib, sem_la, sem_lb, sem_z):
        c = lax.axis_index("c")
        s = lax.axis_index("s")
        rows_per_tile = N_PAD // NS
        my_rows = pl.ds(s * rows_per_tile, rows_per_tile)
        zc = pltpu.async_copy(zeros_hbm.at[my_rows], acc_sh.at[my_rows],
                              sem_z)

        base = c * (EH // NC) + s * EPW

        def off(j):
            return base + jnp.minimum(j, NCHUNK_S - 1) * SCH

        def fire_idx(j, ib, sem):
            return pltpu.async_copy(r_hbm.at[pl.ds(e0 + off(j), SCH)], ib, sem)

        def fire_load(j, rb, sem):
            return pltpu.async_copy(upd_hbm.at[pl.ds(off(j), SCH)], rb, sem)

        def wait_ib():
            pltpu.make_async_copy(r_hbm.at[pl.ds(0, SCH)], idx_b, sem_ib).wait()

        def wait_lb():
            pltpu.make_async_copy(upd_hbm.at[pl.ds(0, SCH)], rows_b,
                                  sem_lb).wait()

        ia = fire_idx(0, idx_a, sem_ia)
        la = fire_load(0, rows_a, sem_la)
        fire_idx(1, idx_b, sem_ib)
        fire_load(1, rows_b, sem_lb)
        zc.wait()
        plsc.subcore_barrier()
        ia.wait()
        la.wait()

        def body(k, carry):
            j2 = 2 * k + 2
            j3 = 2 * k + 3
            pltpu.sync_copy(rows_a, acc_sh.at[idx_a], add=True)
            ia2 = fire_idx(j2, idx_a, sem_ia)
            la2 = fire_load(j2, rows_a, sem_la)
            wait_ib()
            wait_lb()
            pltpu.sync_copy(rows_b, acc_sh.at[idx_b], add=True)
            fire_idx(j3, idx_b, sem_ib)
            fire_load(j3, rows_b, sem_lb)
            ia2.wait()
            la2.wait()
            return carry

        lax.fori_loop(0, NPAIR_S, body, 0)

        pltpu.sync_copy(rows_a, acc_sh.at[idx_a], add=True)
        wait_ib()
        wait_lb()

        plsc.subcore_barrier()
        pltpu.sync_copy(acc_sh.at[my_rows], agg_hbm.at[c].at[my_rows])

    return scatter_k


_gather_h1 = _make_gather(0)
_gather_h2 = _make_gather(EH)
_scatter_h1 = _make_scatter(0)
_scatter_h2 = _make_scatter(EH)


def kernel(node_features, edge_features, senders, receivers,
           We1, be1, We2, be2, ln_e_scale, ln_e_bias,
           Wn1, bn1, Wn2, bn2, ln_n_scale, ln_n_bias):
    s32 = senders.astype(jnp.int32)
    r32 = receivers.astype(jnp.int32)

    ztbl = _proj(node_features, We1[:D], We1[D:2 * D])
    idx_cat = jnp.concatenate([s32, r32])
    w1c = We1[2 * D:]
    eb = (w1c, be1.reshape(1, D), We2, be2.reshape(1, D),
          ln_e_scale.reshape(1, D), ln_e_bias.reshape(1, D))

    g1 = _gather_h1(ztbl, idx_cat)
    g2 = _gather_h2(ztbl, idx_cat)
    upd1, new_v1 = _edge_mlp1(g1, edge_features, *eb)
    upd2, new_edge = _edge_mlp2(new_v1, g2, edge_features, *eb)

    zeros = jnp.zeros((N_PAD, D), jnp.float32)
    agg_a = _scatter_h1(upd1, r32, zeros)
    agg_b = _scatter_h2(upd2, r32, zeros)

    new_node = _node_mlp(
        node_features, agg_a, agg_b,
        Wn1[:D], Wn1[D:], bn1.reshape(1, D), Wn2, bn2.reshape(1, D),
        ln_n_scale.reshape(1, D), ln_n_bias.reshape(1, D))
    return new_node, new_edge


# speedup vs baseline: 1.0430x; 1.0035x over previous
import functools

import jax
import jax.numpy as jnp
from jax import lax
from jax.experimental import pallas as pl
from jax.experimental.pallas import tpu as pltpu
from jax.experimental.pallas import tpu_sc as plsc

N = 10000
E = 320000
D = 128
N_PAD = 10240

NC, NS = 2, 16
NW = NC * NS
EH = E // 2
EPW = EH // NW

EPT = EH // NS
GCH = 128
NCHUNK_G = (EPT + GCH - 1) // GCH
NPAIR_G = (NCHUNK_G - 1) // 2

SCH = 40
NCHUNK_S = EPW // SCH
NPAIR_S = (NCHUNK_S - 1) // 2

NODE_BLK = 5000
EDGE_BLK = 8000
HBLKS = EH // EDGE_BLK

_mesh = plsc.VectorSubcoreMesh(core_axis_name="c", subcore_axis_name="s")


def _proj_body(nf_ref, wa_ref, wb_ref, z_ref):
    nf = nf_ref[...]
    z_ref[0] = jnp.dot(nf, wa_ref[...], preferred_element_type=jnp.float32)
    z_ref[1] = jnp.dot(nf, wb_ref[...], preferred_element_type=jnp.float32)


def _edge_math(gs, gr, ef, w1, b1, w2, b2, sc, bi):
    pre = (gs + gr + b1
           + jnp.dot(ef, w1, preferred_element_type=jnp.float32))
    h = jnp.maximum(pre, 0.0)
    y = jnp.dot(h, w2, preferred_element_type=jnp.float32) + b2
    mean = jnp.mean(y, axis=-1, keepdims=True)
    var = jnp.mean((y - mean) ** 2, axis=-1, keepdims=True)
    upd = (y - mean) * lax.rsqrt(var + 1e-5) * sc + bi
    return upd, upd + ef


def _edge_body1(gs_ref, gr_ref, ef_ref, w1_ref, b1_ref, w2_ref, b2_ref,
                sc_ref, bi_ref, upd_ref, new_ref):
    upd, new = _edge_math(gs_ref[0], gr_ref[0], ef_ref[...], w1_ref[...],
                          b1_ref[...], w2_ref[...], b2_ref[...], sc_ref[...],
                          bi_ref[...])
    upd_ref[...] = upd
    new_ref[...] = new


def _edge_body2(alias_ref, gs_ref, gr_ref, ef_ref, w1_ref, b1_ref, w2_ref,
                b2_ref, sc_ref, bi_ref, upd_ref, new_ref):
    del alias_ref
    upd, new = _edge_math(gs_ref[0], gr_ref[0], ef_ref[...], w1_ref[...],
                          b1_ref[...], w2_ref[...], b2_ref[...], sc_ref[...],
                          bi_ref[...])
    upd_ref[...] = upd
    new_ref[...] = new


def _node_body(nf_ref, a0_ref, a1_ref, a2_ref, a3_ref, w1a_ref, w1b_ref,
               b1_ref, w2_ref, b2_ref, sc_ref, bi_ref, out_ref):
    nf = nf_ref[...]
    agg = (a0_ref[0] + a1_ref[0]) + (a2_ref[0] + a3_ref[0])
    pre = (jnp.dot(nf, w1a_ref[...], preferred_element_type=jnp.float32)
           + jnp.dot(agg, w1b_ref[...], preferred_element_type=jnp.float32)
           + b1_ref[...])
    h = jnp.maximum(pre, 0.0)
    y = jnp.dot(h, w2_ref[...], preferred_element_type=jnp.float32) + b2_ref[...]
    mean = jnp.mean(y, axis=-1, keepdims=True)
    var = jnp.mean((y - mean) ** 2, axis=-1, keepdims=True)
    out_ref[...] = ((y - mean) * lax.rsqrt(var + 1e-5) * sc_ref[...]
                    + bi_ref[...] + nf)


def _row_spec(blk):
    return pl.BlockSpec((blk, D), lambda i: (i, 0))


def _full_spec(shape):
    return pl.BlockSpec(shape, lambda i: tuple(0 for _ in shape))


def _proj(nf, wa, wb):
    blk = 2048
    return pl.pallas_call(
        _proj_body,
        grid=(N_PAD // blk,),
        in_specs=[_row_spec(blk), _full_spec((D, D)), _full_spec((D, D))],
        out_specs=pl.BlockSpec((2, blk, D), lambda i: (0, i, 0)),
        out_shape=jax.ShapeDtypeStruct((2, N_PAD, D), jnp.float32),
    )(nf, wa, wb)


def _half_spec(half):
    blk0 = half * HBLKS
    return pl.BlockSpec((EDGE_BLK, D), lambda i: (blk0 + i, 0))


def _g_spec(plane):
    return pl.BlockSpec((1, EDGE_BLK, D), lambda i: (plane, i, 0))


def _edge_mlp1(gcat, ef, w1, b1, w2, b2, sc, bi):
    wspecs = [_full_spec((D, D)), _full_spec((1, D)), _full_spec((D, D)),
              _full_spec((1, D)), _full_spec((1, D)), _full_spec((1, D))]
    return pl.pallas_call(
        _edge_body1,
        grid=(HBLKS,),
        in_specs=[_g_spec(0), _g_spec(1), _half_spec(0)] + wspecs,
        out_specs=[_row_spec(EDGE_BLK), _half_spec(0)],
        out_shape=[jax.ShapeDtypeStruct((EH, D), jnp.float32),
                   jax.ShapeDtypeStruct((E, D), jnp.float32)],
    )(gcat, gcat, ef, w1, b1, w2, b2, sc, bi)


def _edge_mlp2(new_prev, gcat, ef, w1, b1, w2, b2, sc, bi):
    wspecs = [_full_spec((D, D)), _full_spec((1, D)), _full_spec((D, D)),
              _full_spec((1, D)), _full_spec((1, D)), _full_spec((1, D))]
    return pl.pallas_call(
        _edge_body2,
        grid=(HBLKS,),
        in_specs=[pl.BlockSpec((8, D), lambda i: (0, 0)),
                  _g_spec(0), _g_spec(1), _half_spec(1)] + wspecs,
        out_specs=[_row_spec(EDGE_BLK), _half_spec(1)],
        out_shape=[jax.ShapeDtypeStruct((EH, D), jnp.float32),
                   jax.ShapeDtypeStruct((E, D), jnp.float32)],
        input_output_aliases={0: 1},
    )(new_prev, gcat, gcat, ef, w1, b1, w2, b2, sc, bi)


def _node_mlp(nf, agg_a, agg_b, w1a, w1b, b1, w2, b2, sc, bi):
    grid = N // NODE_BLK
    aspec = [pl.BlockSpec((1, NODE_BLK, D), lambda i: (0, i, 0)),
             pl.BlockSpec((1, NODE_BLK, D), lambda i: (1, i, 0))]
    return pl.pallas_call(
        _node_body,
        grid=(grid,),
        in_specs=[_row_spec(NODE_BLK)] + aspec + aspec
        + [_full_spec((D, D)), _full_spec((D, D)), _full_spec((1, D)),
           _full_spec((D, D)), _full_spec((1, D)), _full_spec((1, D)),
           _full_spec((1, D))],
        out_specs=_row_spec(NODE_BLK),
        out_shape=jax.ShapeDtypeStruct((N, D), jnp.float32),
    )(nf, agg_a, agg_a, agg_b, agg_b, w1a, w1b, b1, w2, b2, sc, bi)


def _make_gather(e0):
    @functools.partial(
        pl.kernel,
        mesh=_mesh,
        out_type=jax.ShapeDtypeStruct((2, EH, D), jnp.float32),
        scratch_types=[
            pltpu.VMEM((GCH,), jnp.int32), pltpu.VMEM((GCH,), jnp.int32),
            pltpu.VMEM((GCH, D), jnp.float32), pltpu.VMEM((GCH, D), jnp.float32),
            pltpu.VMEM_SHARED((N_PAD, D), jnp.float32),
            pltpu.SemaphoreType.DMA, pltpu.SemaphoreType.DMA,
            pltpu.SemaphoreType.DMA, pltpu.SemaphoreType.DMA,
            pltpu.SemaphoreType.DMA, pltpu.SemaphoreType.DMA,
        ],
    )
    def gather_k(ztbl_hbm, idx_hbm, g_hbm,
                 idx_a, idx_b, rows_a, rows_b, spm_tbl,
                 sem_ia, sem_ib, sem_ga, sem_gb, sem_wa, sem_wb):
        c = lax.axis_index("c")
        s = lax.axis_index("s")
        rows_per_tile = N_PAD // NS
        my_rows = pl.ds(s * rows_per_tile, rows_per_tile)
        pltpu.sync_copy(ztbl_hbm.at[c].at[my_rows], spm_tbl.at[my_rows])
        plsc.subcore_barrier()

        base = s * EPT

        def off(j):
            return base + jnp.minimum(j * GCH, EPT - GCH)

        def fire_idx(j, ib, sem):
            return pltpu.async_copy(
                idx_hbm.at[pl.ds(c * E + e0 + off(j), GCH)], ib, sem)

        def fire_gather(ib, rb, sem):
            return pltpu.async_copy(spm_tbl.at[ib], rb, sem)

        def fire_write(j, rb, sem):
            return pltpu.async_copy(rb, g_hbm.at[c].at[pl.ds(off(j), GCH)],
                                    sem)

        def i_b_mk():
            return pltpu.make_async_copy(idx_hbm.at[pl.ds(0, GCH)],
                                         idx_b, sem_ib)

        def g_a_mk():
            return pltpu.make_async_copy(spm_tbl.at[idx_a], rows_a, sem_ga)

        fire_idx(0, idx_a, sem_ia).wait()
        fire_gather(idx_a, rows_a, sem_ga)
        fire_idx(1, idx_b, sem_ib)

        def body(k, carry):
            j0 = 2 * k
            j1 = j0 + 1
            j2 = j0 + 2
            j3 = j0 + 3
            i_b_mk().wait()
            g_b = fire_gather(idx_b, rows_b, sem_gb)
            g_a_mk().wait()
            w_a = fire_write(j0, rows_a, sem_wa)
            i_a = fire_idx(j2, idx_a, sem_ia)
            g_b.wait()
            w_a.wait()
            w_b = fire_write(j1, rows_b, sem_wb)
            i_a.wait()
            fire_gather(idx_a, rows_a, sem_ga)
            fire_idx(j3, idx_b, sem_ib)
            w_b.wait()
            return carry

        lax.fori_loop(0, NPAIR_G, body, 0)

        g_a_mk().wait()
        w_last = fire_write(NCHUNK_G - 1, rows_a, sem_wa)
        i_b_mk().wait()
        w_last.wait()

    return gather_k


def _make_scatter(e0):
    @functools.partial(
        pl.kernel,
        mesh=_mesh,
        out_type=jax.ShapeDtypeStruct((NC, N_PAD, D), jnp.float32),
        scratch_types=[
            pltpu.VMEM((SCH,), jnp.int32), pltpu.VMEM((SCH,), jnp.int32),
            pltpu.VMEM((SCH, D), jnp.float32), pltpu.VMEM((SCH, D), jnp.float32),
            pltpu.VMEM_SHARED((N_PAD, D), jnp.float32),
            pltpu.SemaphoreType.DMA, pltpu.SemaphoreType.DMA,
            pltpu.SemaphoreType.DMA, pltpu.SemaphoreType.DMA,
            pltpu.SemaphoreType.DMA,
        ],
    )
    def scatter_k(upd_hbm, r_hbm, zeros_hbm, agg_hbm,
                  idx_a, idx_b, rows_a, rows_b, acc_sh,
                  sem_ia, sem_ib, sem_la, sem_lb, sem_z):
        c = lax.axis_index("c")
        s = lax.axis_index("s")
        rows_per_tile = N_PAD // NS
        my_rows = pl.ds(s * rows_per_tile, rows_per_tile)
        zc = pltpu.async_copy(zeros_hbm.at[my_rows], acc_sh.at[my_rows],
                              sem_z)

        base = c * (EH // NC) + s * EPW

        def off(j):
            return base + jnp.minimum(j, NCHUNK_S - 1) * SCH

        def fire_idx(j, ib, sem):
            return pltpu.async_copy(r_hbm.at[pl.ds(e0 + off(j), SCH)], ib, sem)

        def fire_load(j, rb, sem):
            return pltpu.async_copy(upd_hbm.at[pl.ds(off(j), SCH)], rb, sem)

        def wait_ib():
            pltpu.make_async_copy(r_hbm.at[pl.ds(0, SCH)], idx_b, sem_ib).wait()

        def wait_lb():
            pltpu.make_async_copy(upd_hbm.at[pl.ds(0, SCH)], rows_b,
                                  sem_lb).wait()

        ia = fire_idx(0, idx_a, sem_ia)
        la = fire_load(0, rows_a, sem_la)
        fire_idx(1, idx_b, sem_ib)
        fire_load(1, rows_b, sem_lb)
        zc.wait()
        plsc.subcore_barrier()
        ia.wait()
        la.wait()

        def body(k, carry):
            j2 = 2 * k + 2
            j3 = 2 * k + 3
            pltpu.sync_copy(rows_a, acc_sh.at[idx_a], add=True)
            ia2 = fire_idx(j2, idx_a, sem_ia)
            la2 = fire_load(j2, rows_a, sem_la)
            wait_ib()
            wait_lb()
            pltpu.sync_copy(rows_b, acc_sh.at[idx_b], add=True)
            fire_idx(j3, idx_b, sem_ib)
            fire_load(j3, rows_b, sem_lb)
            ia2.wait()
            la2.wait()
            return carry

        lax.fori_loop(0, NPAIR_S, body, 0)

        pltpu.sync_copy(rows_a, acc_sh.at[idx_a], add=True)
        wait_ib()
        wait_lb()

        plsc.subcore_barrier()
        pltpu.sync_copy(acc_sh.at[my_rows], agg_hbm.at[c].at[my_rows])

    return scatter_k


_gather_h1 = _make_gather(0)
_gather_h2 = _make_gather(EH)
_scatter_h1 = _make_scatter(0)
_scatter_h2 = _make_scatter(EH)


def kernel(node_features, edge_features, senders, receivers,
           We1, be1, We2, be2, ln_e_scale, ln_e_bias,
           Wn1, bn1, Wn2, bn2, ln_n_scale, ln_n_bias):
    s32 = senders.astype(jnp.int32)
    r32 = receivers.astype(jnp.int32)

    ztbl = _proj(node_features, We1[:D], We1[D:2 * D])
    idx_cat = jnp.concatenate([s32, r32])
    w1c = We1[2 * D:]
    eb = (w1c, be1.reshape(1, D), We2, be2.reshape(1, D),
          ln_e_scale.reshape(1, D), ln_e_bias.reshape(1, D))

    g1 = _gather_h1(ztbl, idx_cat)
    g2 = _gather_h2(ztbl, idx_cat)
    upd1, new_v1 = _edge_mlp1(g1, edge_features, *eb)
    upd2, new_edge = _edge_mlp2(new_v1, g2, edge_features, *eb)

    zeros = jnp.zeros((N_PAD, D), jnp.float32)
    agg_a = _scatter_h1(upd1, r32, zeros)
    agg_b = _scatter_h2(upd2, r32, zeros)

    new_node = _node_mlp(
        node_features, agg_a, agg_b,
        Wn1[:D], Wn1[D:], bn1.reshape(1, D), Wn2, bn2.reshape(1, D),
        ln_n_scale.reshape(1, D), ln_n_bias.reshape(1, D))
    return new_node, new_edge
